# baseline (speedup 1.0000x reference)
import functools

import jax
import jax.numpy as jnp
from jax import lax
from jax.experimental import pallas as pl
from jax.experimental.pallas import tpu as pltpu
from jax.experimental.pallas import tpu_sc as plsc

_N, _D = 10000, 128
_DH = _D // 2
_NC, _NS = 2, 16
_CHUNK = 128
_NCH = 2496
_CPT = _NCH // _NS
_SLAB = 26
_RPT = 624
_BN_EPS = 1e-5


def _sc_agg(xv, z, ii, em_r, em_c, lm_r, lm_c):
    mesh = plsc.VectorSubcoreMesh(core_axis_name="c", subcore_axis_name="s")

    @functools.partial(
        pl.kernel,
        out_type=jax.ShapeDtypeStruct((_N, _D), jnp.float32),
        mesh=mesh,
        compiler_params=pltpu.CompilerParams(use_tc_tiling_on_sc=False),
        scratch_types=[
            pltpu.VMEM((_SLAB, _CHUNK), jnp.int32),
            pltpu.VMEM((_SLAB, _CHUNK), jnp.int32),
            pltpu.VMEM((_SLAB, _CHUNK), jnp.int32),
            pltpu.VMEM((1, _CHUNK), jnp.int32),
            pltpu.VMEM((1, _CHUNK), jnp.int32),
            pltpu.VMEM((5, _CHUNK), jnp.int32),
            pltpu.VMEM((_CHUNK, _DH), jnp.float32),
            pltpu.VMEM((_CHUNK, _DH), jnp.float32),
            pltpu.VMEM((_CHUNK, _DH), jnp.float32),
            pltpu.VMEM((_CHUNK, _DH), jnp.float32),
            pltpu.VMEM_SHARED((_N, _DH), jnp.float32),
            pltpu.VMEM_SHARED((_N, _DH), jnp.float32),
            pltpu.SemaphoreType.DMA,
            pltpu.SemaphoreType.DMA,
            pltpu.SemaphoreType.DMA,
            pltpu.SemaphoreType.DMA,
            pltpu.SemaphoreType.DMA,
            pltpu.SemaphoreType.DMA,
            pltpu.SemaphoreType.DMA,
            pltpu.SemaphoreType.DMA,
        ],
    )
    def k(xv_hbm, z_hbm, ii_hbm, emr_hbm, emc_hbm, lmr_hbm, lmc_hbm, out_hbm,
          idx_r, idx_c, idx_d, lx_r, lx_c, iidx, b0, b1, b2, b3, agg, xc,
          g0, g1, g2, g3, s0, s1, s2, s3):
        c = lax.axis_index("c")
        s = lax.axis_index("s")
        gb = (b0, b1, b2, b3)
        gs = (g0, g1, g2, g3)
        ss = (s0, s1, s2, s3)
        gb_init = (b0, b1)
        gs_init = (g0, g1)

        off = (0, 128, 256, 384, 497)
        pltpu.sync_copy(ii_hbm.at[c].at[pl.ds(s * 5, 5)], iidx)
        dprev = pltpu.async_copy(xv_hbm.at[iidx.at[0]], b0, g0)
        for t in range(5):
            dcur = dprev
            if t < 4:
                dprev = pltpu.async_copy(
                    xv_hbm.at[iidx.at[t + 1]],
                    gb_init[(t + 1) % 2], gs_init[(t + 1) % 2])
            dcur.wait()
            pltpu.sync_copy(gb_init[t % 2],
                            xc.at[pl.ds(s * 625 + off[t], _CHUNK)])
        pltpu.sync_copy(z_hbm.at[pl.ds(s * _RPT, _RPT)],
                        agg.at[pl.ds(s * _RPT, _RPT)])

        @pl.when(s < 2)
        def _():
            base = _NS * _RPT + s * 8
            pltpu.sync_copy(z_hbm.at[pl.ds(base, 8)], agg.at[pl.ds(base, 8)])

        @pl.when(s < 4)
        def _():
            pltpu.sync_copy(lmr_hbm.at[pl.ds(s, 1)], lx_r)
            pltpu.sync_copy(lmc_hbm.at[c].at[pl.ds(s, 1)], lx_c)

        plsc.subcore_barrier()

        def G(j, k_):
            if k_ == 0:
                pltpu.async_copy(xc.at[idx_c.at[j]], gb[k_], gs[k_])
            else:
                pltpu.async_copy(xv_hbm.at[idx_d.at[j]], gb[k_], gs[k_])

        def S(j, k_):
            pltpu.async_copy(gb[k_], agg.at[idx_r.at[j]], ss[k_], add=True)

        def Wg(k_):
            pltpu.make_async_copy(xv_hbm.at[pl.ds(0, _CHUNK)], gb[k_],
                                  gs[k_]).wait()

        def Ws(k_):
            pltpu.make_async_copy(gb[k_], agg.at[pl.ds(0, _CHUNK)],
                                  ss[k_]).wait()

        grp = (_SLAB - 4) // 4
        rem = (_SLAB - 4) % 4
        for slab in range(_CPT // _SLAB):
            base = s * _CPT + slab * _SLAB
            pltpu.sync_copy(emr_hbm.at[pl.ds(base, _SLAB)], idx_r)
            pltpu.sync_copy(emc_hbm.at[pl.ds(base, _SLAB)], idx_c)

            def dbl(i, carry):
                r = i // 8
                l = (i % 8) * 16
                idx_d[r, pl.ds(l, 16)] = idx_c[r, pl.ds(l, 16)] * 2 + c
                return carry

            lax.fori_loop(0, _SLAB * 8, dbl, 0)

            G(0, 0)
            G(1, 1)
            Wg(0)
            S(0, 0)
            G(2, 2)
            Wg(1)
            S(1, 1)
            G(3, 3)

            def steady(g, carry):
                for k_ in range(4):
                    j = 4 + g * 4 + k_
                    Ws(k_)
                    Wg((k_ + 2) % 4)
                    S(j - 2, (k_ + 2) % 4)
                    G(j, k_)
                return carry

            lax.fori_loop(0, grp, steady, 0)
            for t in range(rem):
                j = 4 + grp * 4 + t
                k_ = j % 4
                Ws(k_)
                Wg((j - 2) % 4)
                S(j - 2, (j - 2) % 4)
                G(j, k_)
            Wg((_SLAB - 2) % 4)
            S(_SLAB - 2, (_SLAB - 2) % 4)
            Wg((_SLAB - 1) % 4)
            S(_SLAB - 1, (_SLAB - 1) % 4)
            Ws(0)
            Ws(1)
            Ws(2)
            Ws(3)

        @pl.when(s < 4)
        def _():
            pltpu.async_copy(xv_hbm.at[lx_c.at[0]], b0, g0).wait()
            pltpu.sync_copy(b0, agg.at[lx_r.at[0]], add=True)

        plsc.subcore_barrier()
        pltpu.sync_copy(agg.at[pl.ds(s * _RPT, _RPT)],
                        out_hbm.at[pl.ds(s * _RPT, _RPT), pl.ds(c * _DH, _DH)])

        @pl.when(s < 2)
        def _():
            base = _NS * _RPT + s * 8
            pltpu.sync_copy(agg.at[pl.ds(base, 8)],
                            out_hbm.at[pl.ds(base, 8), pl.ds(c * _DH, _DH)])

    return k(xv, z, ii, em_r, em_c, lm_r, lm_c)


def _tc_finish(x, p, eps11, W1, b1, g1, be1, W2, b2, g2, be2):
    def body(x_ref, p_ref, eps_ref, w1_ref, b1_ref, g1_ref, be1_ref,
             w2_ref, b2_ref, g2_ref, be2_ref, o_ref):
        eps = eps_ref[0, 0]
        h = eps * x_ref[...] + p_ref[...]
        h = jnp.dot(h, w1_ref[...], preferred_element_type=jnp.float32) + b1_ref[...]
        m = jnp.mean(h, axis=0, keepdims=True)
        v = jnp.mean((h - m) * (h - m), axis=0, keepdims=True)
        h = (h - m) * lax.rsqrt(v + _BN_EPS) * g1_ref[...] + be1_ref[...]
        h = jnp.maximum(h, 0.0)
        h = jnp.dot(h, w2_ref[...], preferred_element_type=jnp.float32) + b2_ref[...]
        m = jnp.mean(h, axis=0, keepdims=True)
        v = jnp.mean((h - m) * (h - m), axis=0, keepdims=True)
        h = (h - m) * lax.rsqrt(v + _BN_EPS) * g2_ref[...] + be2_ref[...]
        o_ref[...] = jnp.maximum(h, 0.0)

    return pl.pallas_call(
        body,
        out_shape=jax.ShapeDtypeStruct((_N, _D), jnp.float32),
    )(x, p, eps11, W1, b1, g1, be1, W2, b2, g2, be2)


def kernel(x, edge_index, eps, W1, b1, g1, be1, W2, b2, g2, be2):
    rowf = edge_index[0]
    colf = edge_index[1]
    em_r = rowf[:_NCH * _CHUNK].reshape(_NCH, _CHUNK)
    em_c = colf[:_NCH * _CHUNK].reshape(_NCH, _CHUNK)
    lm_r = rowf[_NCH * _CHUNK:].reshape(4, _CHUNK)
    lcc = colf[_NCH * _CHUNK:].reshape(4, _CHUNK) * 2
    lm_c = jnp.stack([lcc, lcc + 1])
    xv = x.reshape(2 * _N, _DH)
    z = jnp.zeros((_N, _DH), jnp.float32)
    r80 = jnp.arange(80, dtype=jnp.int32)
    tsel = r80 % 5
    tbase = (r80 // 5) * 625 + jnp.where(tsel < 4, tsel * 128, 497)
    node = tbase[:, None] + jnp.arange(_CHUNK, dtype=jnp.int32)[None, :]
    ii = jnp.stack([2 * node, 2 * node + 1])
    p = _sc_agg(xv, z, ii, em_r, em_c, lm_r, lm_c)
    eps11 = jnp.reshape(eps + 1.0, (1, 1)).astype(jnp.float32)
    return _tc_finish(
        x, p, eps11,
        W1, b1.reshape(1, _D), g1.reshape(1, _D), be1.reshape(1, _D),
        W2, b2.reshape(1, _D), g2.reshape(1, _D), be2.reshape(1, _D))


# speedup vs baseline: 1.1387x; 1.1387x over previous
import functools

import jax
import jax.numpy as jnp
from jax import lax
from jax.experimental import pallas as pl
from jax.experimental.pallas import tpu as pltpu
from jax.experimental.pallas import tpu_sc as plsc

_N, _D = 10000, 128
_DH = _D // 2
_NC, _NS = 2, 16
_CHUNK = 128
_NCH = 2496
_CPT = _NCH // _NS
_RPT = 624
_PAGG = _N // 2
_BN_EPS = 1e-5


def _sc_agg(xv, z, em_r, em_c, lm_r, lm_c):
    mesh = plsc.VectorSubcoreMesh(core_axis_name="c", subcore_axis_name="s")

    @functools.partial(
        pl.kernel,
        out_type=jax.ShapeDtypeStruct((_N, _D), jnp.float32),
        mesh=mesh,
        compiler_params=pltpu.CompilerParams(use_tc_tiling_on_sc=False),
        scratch_types=[
            pltpu.VMEM((_CPT, _CHUNK), jnp.int32),
            pltpu.VMEM((_CPT, _CHUNK), jnp.int32),
            pltpu.VMEM((1, _CHUNK), jnp.int32),
            pltpu.VMEM((1, _CHUNK), jnp.int32),
            pltpu.VMEM((_CHUNK, _DH), jnp.float32),
            pltpu.VMEM((_CHUNK, _DH), jnp.float32),
            pltpu.VMEM((_CHUNK, _DH), jnp.float32),
            pltpu.VMEM((_CHUNK, _DH), jnp.float32),
            pltpu.VMEM_SHARED((_N, _DH), jnp.float32),
            pltpu.SemaphoreType.DMA,
            pltpu.SemaphoreType.DMA,
            pltpu.SemaphoreType.DMA,
            pltpu.SemaphoreType.DMA,
            pltpu.SemaphoreType.DMA,
            pltpu.SemaphoreType.DMA,
            pltpu.SemaphoreType.DMA,
            pltpu.SemaphoreType.DMA,
        ],
    )
    def k(xv_hbm, z_hbm, emr_hbm, emc_hbm, lmr_hbm, lmc_hbm, out_hbm,
          idx_r, idx_c, lx_r, lx_c, b0, b1, b2, b3, agg,
          g0, g1, g2, g3, s0, s1, s2, s3):
        c = lax.axis_index("c")
        s = lax.axis_index("s")
        xh = xv_hbm
        aggv = agg
        gb = (b0, b1, b2, b3)
        gs = (g0, g1, g2, g3)
        ss = (s0, s1, s2, s3)

        pltpu.sync_copy(emr_hbm.at[pl.ds(s * _CPT, _CPT)], idx_r)
        pltpu.sync_copy(emc_hbm.at[c].at[pl.ds(s * _CPT, _CPT)], idx_c)

        @pl.when(s < 4)
        def _():
            pltpu.sync_copy(lmr_hbm.at[pl.ds(s, 1)], lx_r)
            pltpu.sync_copy(lmc_hbm.at[c].at[pl.ds(s, 1)], lx_c)

        pltpu.sync_copy(z_hbm.at[pl.ds(s * _RPT, _RPT)],
                        agg.at[pl.ds(s * _RPT, _RPT)])

        @pl.when(s < 2)
        def _():
            base = _NS * _RPT + s * 8
            pltpu.sync_copy(z_hbm.at[pl.ds(base, 8)], agg.at[pl.ds(base, 8)])

        plsc.subcore_barrier()

        def G(j, k_):
            pltpu.async_copy(xh.at[idx_c.at[j]], gb[k_], gs[k_])

        def S(j, k_):
            pltpu.async_copy(gb[k_], aggv.at[idx_r.at[j]], ss[k_], add=True)

        def Wg(k_):
            pltpu.make_async_copy(xh.at[pl.ds(0, _CHUNK)], gb[k_], gs[k_]).wait()

        def Ws(k_):
            pltpu.make_async_copy(gb[k_], aggv.at[pl.ds(0, _CHUNK)], ss[k_]).wait()

        G(0, 0)
        G(1, 1)
        Wg(0)
        S(0, 0)
        G(2, 2)
        Wg(1)
        S(1, 1)
        G(3, 3)

        def steady(g, carry):
            for k_ in range(4):
                j = 4 + g * 4 + k_
                Ws(k_)
                Wg((k_ + 2) % 4)
                S(j - 2, (k_ + 2) % 4)
                G(j, k_)
            return carry

        lax.fori_loop(0, (_CPT - 4) // 4, steady, 0)
        Wg(2)
        S(_CPT - 2, 2)
        Wg(3)
        S(_CPT - 1, 3)
        Ws(0)
        Ws(1)
        Ws(2)
        Ws(3)

        @pl.when(s < 4)
        def _():
            pltpu.async_copy(xh.at[lx_c.at[0]], b0, g0).wait()
            pltpu.sync_copy(b0, aggv.at[lx_r.at[0]], add=True)

        plsc.subcore_barrier()
        pltpu.sync_copy(aggv.at[pl.ds(s * _RPT, _RPT)],
                        out_hbm.at[pl.ds(s * _RPT, _RPT), pl.ds(c * _DH, _DH)])

        @pl.when(s < 2)
        def _():
            base = _NS * _RPT + s * 8
            pltpu.sync_copy(aggv.at[pl.ds(base, 8)],
                            out_hbm.at[pl.ds(base, 8), pl.ds(c * _DH, _DH)])

    return k(xv, z, em_r, em_c, lm_r, lm_c)


def _tc_finish(x, p, eps11, W1, b1, g1, be1, W2, b2, g2, be2):
    def body(x_ref, p_ref, eps_ref, w1_ref, b1_ref, g1_ref, be1_ref,
             w2_ref, b2_ref, g2_ref, be2_ref, o_ref):
        eps = eps_ref[0, 0]
        h = eps * x_ref[...] + p_ref[...]
        h = jnp.dot(h, w1_ref[...], preferred_element_type=jnp.float32) + b1_ref[...]
        m = jnp.mean(h, axis=0, keepdims=True)
        v = jnp.mean((h - m) * (h - m), axis=0, keepdims=True)
        h = (h - m) * lax.rsqrt(v + _BN_EPS) * g1_ref[...] + be1_ref[...]
        h = jnp.maximum(h, 0.0)
        h = jnp.dot(h, w2_ref[...], preferred_element_type=jnp.float32) + b2_ref[...]
        m = jnp.mean(h, axis=0, keepdims=True)
        v = jnp.mean((h - m) * (h - m), axis=0, keepdims=True)
        h = (h - m) * lax.rsqrt(v + _BN_EPS) * g2_ref[...] + be2_ref[...]
        o_ref[...] = jnp.maximum(h, 0.0)

    return pl.pallas_call(
        body,
        out_shape=jax.ShapeDtypeStruct((_N, _D), jnp.float32),
    )(x, p, eps11, W1, b1, g1, be1, W2, b2, g2, be2)


def kernel(x, edge_index, eps, W1, b1, g1, be1, W2, b2, g2, be2):
    rowf = edge_index[0]
    colf = edge_index[1]
    em_r = rowf[:_NCH * _CHUNK].reshape(_NCH, _CHUNK)
    cc = colf[:_NCH * _CHUNK].reshape(_NCH, _CHUNK) * 2
    em_c = jnp.stack([cc, cc + 1])
    lm_r = rowf[_NCH * _CHUNK:].reshape(4, _CHUNK)
    lcc = colf[_NCH * _CHUNK:].reshape(4, _CHUNK) * 2
    lm_c = jnp.stack([lcc, lcc + 1])
    xv = x.reshape(2 * _N, _DH)
    z = jnp.zeros((_N, _DH), jnp.float32)
    p = _sc_agg(xv, z, em_r, em_c, lm_r, lm_c)
    eps11 = jnp.reshape(eps + 1.0, (1, 1)).astype(jnp.float32)
    return _tc_finish(
        x, p, eps11,
        W1, b1.reshape(1, _D), g1.reshape(1, _D), be1.reshape(1, _D),
        W2, b2.reshape(1, _D), g2.reshape(1, _D), be2.reshape(1, _D))
